# baseline (speedup 1.0000x reference)
import functools

import jax
import jax.numpy as jnp
from jax import lax
from jax.experimental import pallas as pl
from jax.experimental.pallas import tpu as pltpu
from jax.experimental.pallas import tpu_sc as plsc

_D = 128
_EB = 128
_IC = 40
_NC = 2
_TILES = 16
_NW = _NC * _TILES
_G = 128


def _make_seg_sum(n_pad, t_b):
    assert t_b % _IC == 0 and _IC % 8 == 0
    rows_t = n_pad // _TILES
    mesh = plsc.VectorSubcoreMesh(core_axis_name="c", subcore_axis_name="s",
                                  num_cores=_NC, num_subcores=_TILES)

    @functools.partial(
        pl.kernel,
        out_type=jax.ShapeDtypeStruct((_NC, n_pad, _D), jnp.float32),
        mesh=mesh,
        scratch_types=[
            pltpu.VMEM_SHARED((n_pad, _D), jnp.float32),
            pltpu.VMEM((_IC, _EB), jnp.int32),
            pltpu.VMEM((_IC, _EB), jnp.int32),
            pltpu.VMEM((2, _EB, _D), jnp.float32),
            pltpu.SemaphoreType.DMA((2,)),
            pltpu.SemaphoreType.DMA((2,)),
        ],
    )
    def seg_sum(h_hbm, src_hbm, dst_hbm, out_hbm,
                acc, src_v, dst_v, rb, sem_g, sem_s):
        c = lax.axis_index("c")
        s = lax.axis_index("s")
        wid = s * _NC + c
        base = s * rows_t
        ebase = wid * t_b

        pltpu.sync_copy(h_hbm.at[pl.ds(base, rows_t)],
                        acc.at[pl.ds(base, rows_t)])
        plsc.subcore_barrier()

        def chunk(ci, carry):
            off = ebase + ci * _IC
            pltpu.sync_copy(src_hbm.at[pl.ds(off, _IC)], src_v)
            pltpu.sync_copy(dst_hbm.at[pl.ds(off, _IC)], dst_v)
            pltpu.async_copy(h_hbm.at[src_v.at[0]], rb.at[0], sem_g.at[0])
            pltpu.make_async_copy(h_hbm.at[src_v.at[0]], rb.at[0],
                                  sem_g.at[0]).wait()
            pltpu.async_copy(rb.at[0], acc.at[dst_v.at[0]], sem_s.at[0],
                             add=True)
            pltpu.async_copy(h_hbm.at[src_v.at[1]], rb.at[1], sem_g.at[1])

            def body(j, carry2):
                pltpu.make_async_copy(h_hbm.at[src_v.at[j]], rb.at[j % 2],
                                      sem_g.at[j % 2]).wait()
                pltpu.async_copy(rb.at[j % 2], acc.at[dst_v.at[j]],
                                 sem_s.at[j % 2], add=True)
                pltpu.make_async_copy(rb.at[(j + 1) % 2],
                                      acc.at[dst_v.at[j - 1]],
                                      sem_s.at[(j + 1) % 2]).wait()
                jn = jnp.minimum(j + 1, _IC - 1)
                pltpu.async_copy(h_hbm.at[src_v.at[jn]], rb.at[(j + 1) % 2],
                                 sem_g.at[(j + 1) % 2])
                return carry2

            lax.fori_loop(1, _IC, body, 0)
            pltpu.make_async_copy(rb.at[(_IC - 1) % 2],
                                  acc.at[dst_v.at[_IC - 1]],
                                  sem_s.at[(_IC - 1) % 2]).wait()
            pltpu.make_async_copy(h_hbm.at[src_v.at[_IC - 1]], rb.at[_IC % 2],
                                  sem_g.at[_IC % 2]).wait()
            return carry

        lax.fori_loop(0, t_b // _IC, chunk, 0)

        plsc.subcore_barrier()
        pltpu.sync_copy(acc.at[pl.ds(base, rows_t)],
                        out_hbm.at[c].at[pl.ds(base, rows_t)])

    return seg_sum


def _mlp(ps, h, w1, w2, n_pad, blk):
    def body(pa_ref, pb_ref, h_ref, w1_ref, w2_ref, o_ref):
        z = pa_ref[0] + pb_ref[0] - h_ref[...]
        z = lax.dot_general(z, w1_ref[...], (((1,), (1,)), ((), ())),
                            preferred_element_type=jnp.float32,
                            precision=lax.Precision.HIGHEST)
        z = 1.0 / (1.0 + jnp.exp(-z))
        z = lax.dot_general(z, w2_ref[...], (((1,), (1,)), ((), ())),
                            preferred_element_type=jnp.float32,
                            precision=lax.Precision.HIGHEST)
        o_ref[...] = 1.0 / (1.0 + jnp.exp(-z))

    return pl.pallas_call(
        body,
        grid=(n_pad // blk,),
        in_specs=[
            pl.BlockSpec((1, blk, _D), lambda i: (0, i, 0)),
            pl.BlockSpec((1, blk, _D), lambda i: (1, i, 0)),
            pl.BlockSpec((blk, _D), lambda i: (i, 0)),
            pl.BlockSpec((_D, _D), lambda i: (0, 0)),
            pl.BlockSpec((_D, _D), lambda i: (0, 0)),
        ],
        out_specs=pl.BlockSpec((blk, _D), lambda i: (i, 0)),
        out_shape=jax.ShapeDtypeStruct((n_pad, _D), jnp.float32),
    )(ps, ps, h, w1, w2)


def _pool_head(h, batch3, w_pad, b3, n_pad, blk, n_cls):
    steps = n_pad // blk
    cpad = w_pad.shape[0]

    def body(h_ref, b_ref, w_ref, bias_ref, logp_ref, xr_ref):
        i = pl.program_id(0)
        bb = b_ref[0, 0, :]
        oh = (bb[:, None] == lax.broadcasted_iota(jnp.int32, (blk, _G), 1)
              ).astype(jnp.float32)
        contrib = lax.dot_general(oh, h_ref[...], (((0,), (0,)), ((), ())),
                                  preferred_element_type=jnp.float32,
                                  precision=lax.Precision.HIGHEST)

        @pl.when(i == 0)
        def _():
            xr_ref[...] = contrib

        @pl.when(i > 0)
        def _():
            xr_ref[...] = xr_ref[...] + contrib

        @pl.when(i == steps - 1)
        def _():
            xr = xr_ref[...]
            logits = lax.dot_general(xr, w_ref[...], (((1,), (1,)), ((), ())),
                                     preferred_element_type=jnp.float32,
                                     precision=lax.Precision.HIGHEST)
            logits = logits + bias_ref[0, 0, :][None, :]
            m = jnp.max(logits, axis=1, keepdims=True)
            ex = jnp.exp(logits - m)
            lse = jnp.log(jnp.sum(ex, axis=1, keepdims=True))
            lp = logits - m - lse
            logp_ref[...] = lp[:, :n_cls]

    return pl.pallas_call(
        body,
        grid=(steps,),
        in_specs=[
            pl.BlockSpec((blk, _D), lambda i: (i, 0)),
            pl.BlockSpec((1, 1, blk), lambda i: (i, 0, 0)),
            pl.BlockSpec((cpad, _D), lambda i: (0, 0)),
            pl.BlockSpec((1, 1, cpad), lambda i: (0, 0, 0)),
        ],
        out_specs=[
            pl.BlockSpec((_G, n_cls), lambda i: (0, 0)),
            pl.BlockSpec((_G, _D), lambda i: (0, 0)),
        ],
        out_shape=(
            jax.ShapeDtypeStruct((_G, n_cls), jnp.float32),
            jax.ShapeDtypeStruct((_G, _D), jnp.float32),
        ),
    )(h, batch3, w_pad, b3)


def kernel(x, edge_index, batch, conv_w, fc1_w, fc1_b):
    n, d = x.shape
    e = edge_index.shape[1]
    n_layers = conv_w.shape[0] // 2
    n_cls = fc1_w.shape[0]
    assert d == _D

    n_pad = -(-n // 128) * 128
    blk = n_pad // 4
    t_b = -(-e // (_NW * _EB))
    t_b = -(-t_b // _IC) * _IC
    e_pad = _NW * _EB * t_b

    src = edge_index[0]
    dst = edge_index[1]
    pad_n = e_pad - e
    fill = jnp.arange(pad_n, dtype=jnp.int32)
    src_p = jnp.concatenate([src, fill % n]).reshape(e_pad // _EB, _EB)
    dst_p = jnp.concatenate([dst, n + fill % (n_pad - n)]).reshape(
        e_pad // _EB, _EB)

    h = jnp.pad(x, ((0, n_pad - n), (0, 0)))
    batch3 = jnp.concatenate(
        [batch, jnp.full((n_pad - n,), _G, jnp.int32)]).reshape(
        n_pad // blk, 1, blk)

    cpad = 16
    w_pad = jnp.pad(fc1_w, ((0, cpad - n_cls), (0, 0)))
    b3 = jnp.pad(fc1_b, (0, cpad - n_cls),
                 constant_values=-1e30).reshape(1, 1, cpad)

    seg_sum = _make_seg_sum(n_pad, t_b)
    for i in range(n_layers):
        ps = seg_sum(h, src_p, dst_p)
        h = _mlp(ps, h, conv_w[2 * i], conv_w[2 * i + 1], n_pad, blk)

    logp, xr = _pool_head(h, batch3, w_pad, b3, n_pad, blk, n_cls)
    return logp, xr


# speedup vs baseline: 1.1615x; 1.1615x over previous
import functools

import jax
import jax.numpy as jnp
from jax import lax
from jax.experimental import pallas as pl
from jax.experimental.pallas import tpu as pltpu
from jax.experimental.pallas import tpu_sc as plsc

_D = 128
_EB = 128
_IC = 40
_NC = 2
_TILES = 16
_NW = _NC * _TILES
_G = 128


def _make_seg_sum(n_pad, t_b):
    assert t_b % _IC == 0 and _IC % 8 == 0
    rows_t = n_pad // _TILES
    mesh = plsc.VectorSubcoreMesh(core_axis_name="c", subcore_axis_name="s",
                                  num_cores=_NC, num_subcores=_TILES)

    @functools.partial(
        pl.kernel,
        out_type=jax.ShapeDtypeStruct((_NC, n_pad, _D), jnp.float32),
        mesh=mesh,
        scratch_types=[
            pltpu.VMEM_SHARED((n_pad, _D), jnp.float32),
            pltpu.VMEM((_IC, _EB), jnp.int32),
            pltpu.VMEM((_IC, _EB), jnp.int32),
            pltpu.VMEM((2, _EB, _D), jnp.float32),
            pltpu.SemaphoreType.DMA((2,)),
        ],
    )
    def seg_sum(hz_hbm, src_hbm, dst_hbm, out_hbm,
                acc, src_v, dst_v, rb, sem):
        c = lax.axis_index("c")
        s = lax.axis_index("s")
        wid = s * _NC + c
        base = s * rows_t
        ebase = wid * t_b

        pltpu.sync_copy(hz_hbm.at[pl.ds(c * n_pad + base, rows_t)],
                        acc.at[pl.ds(base, rows_t)])
        plsc.subcore_barrier()

        def chunk(ci, carry):
            off = ebase + ci * _IC
            pltpu.sync_copy(src_hbm.at[pl.ds(off, _IC)], src_v)
            pltpu.sync_copy(dst_hbm.at[pl.ds(off, _IC)], dst_v)
            pltpu.async_copy(hz_hbm.at[src_v.at[0]], rb.at[0], sem.at[0])

            def body(j, carry2):
                jn = jnp.minimum(j + 1, _IC - 1)
                pltpu.async_copy(hz_hbm.at[src_v.at[jn]], rb.at[(j + 1) % 2],
                                 sem.at[(j + 1) % 2])
                pltpu.make_async_copy(hz_hbm.at[src_v.at[j]], rb.at[j % 2],
                                      sem.at[j % 2]).wait()
                pltpu.sync_copy(rb.at[j % 2], acc.at[dst_v.at[j]], add=True)
                return carry2

            lax.fori_loop(0, _IC, body, 0)
            pltpu.make_async_copy(hz_hbm.at[src_v.at[_IC - 1]], rb.at[_IC % 2],
                                  sem.at[_IC % 2]).wait()
            return carry

        lax.fori_loop(0, t_b // _IC, chunk, 0)

        plsc.subcore_barrier()
        pltpu.sync_copy(acc.at[pl.ds(base, rows_t)],
                        out_hbm.at[c].at[pl.ds(base, rows_t)])

    return seg_sum


def _mlp(ps, hz, w1, w2, n_pad, blk):
    def body(pa_ref, pb_ref, hz_ref, w1_ref, w2_ref, o_ref):
        del hz_ref
        z = pa_ref[0] + pb_ref[0]
        z = lax.dot_general(z, w1_ref[...], (((1,), (1,)), ((), ())),
                            preferred_element_type=jnp.float32,
                            precision=lax.Precision.HIGHEST)
        z = 1.0 / (1.0 + jnp.exp(-z))
        z = lax.dot_general(z, w2_ref[...], (((1,), (1,)), ((), ())),
                            preferred_element_type=jnp.float32,
                            precision=lax.Precision.HIGHEST)
        o_ref[...] = 1.0 / (1.0 + jnp.exp(-z))

    return pl.pallas_call(
        body,
        grid=(n_pad // blk,),
        in_specs=[
            pl.BlockSpec((1, blk, _D), lambda i: (0, i, 0)),
            pl.BlockSpec((1, blk, _D), lambda i: (1, i, 0)),
            pl.BlockSpec(memory_space=pl.ANY),
            pl.BlockSpec((_D, _D), lambda i: (0, 0)),
            pl.BlockSpec((_D, _D), lambda i: (0, 0)),
        ],
        out_specs=pl.BlockSpec((blk, _D), lambda i: (i, 0)),
        out_shape=jax.ShapeDtypeStruct((2 * n_pad, _D), jnp.float32),
        input_output_aliases={2: 0},
    )(ps, ps, hz, w1, w2)


def _mlp_pool_head(ps, batch3, w1, w2, w_pad, b3, n_pad, blk, n_cls):
    steps = n_pad // blk
    cpad = w_pad.shape[0]

    def body(pa_ref, pb_ref, b_ref, w1_ref, w2_ref, w_ref, bias_ref,
             logp_ref, xr_ref):
        i = pl.program_id(0)
        z = pa_ref[0] + pb_ref[0]
        z = lax.dot_general(z, w1_ref[...], (((1,), (1,)), ((), ())),
                            preferred_element_type=jnp.float32,
                            precision=lax.Precision.HIGHEST)
        z = 1.0 / (1.0 + jnp.exp(-z))
        z = lax.dot_general(z, w2_ref[...], (((1,), (1,)), ((), ())),
                            preferred_element_type=jnp.float32,
                            precision=lax.Precision.HIGHEST)
        h = 1.0 / (1.0 + jnp.exp(-z))

        bb = b_ref[0, 0, :]
        oh = (bb[:, None] == lax.broadcasted_iota(jnp.int32, (blk, _G), 1)
              ).astype(jnp.float32)
        contrib = lax.dot_general(oh, h, (((0,), (0,)), ((), ())),
                                  preferred_element_type=jnp.float32,
                                  precision=lax.Precision.HIGHEST)

        @pl.when(i == 0)
        def _():
            xr_ref[...] = contrib

        @pl.when(i > 0)
        def _():
            xr_ref[...] = xr_ref[...] + contrib

        @pl.when(i == steps - 1)
        def _():
            xr = xr_ref[...]
            logits = lax.dot_general(xr, w_ref[...], (((1,), (1,)), ((), ())),
                                     preferred_element_type=jnp.float32,
                                     precision=lax.Precision.HIGHEST)
            logits = logits + bias_ref[0, 0, :][None, :]
            m = jnp.max(logits, axis=1, keepdims=True)
            ex = jnp.exp(logits - m)
            lse = jnp.log(jnp.sum(ex, axis=1, keepdims=True))
            lp = logits - m - lse
            logp_ref[...] = lp[:, :n_cls]

    return pl.pallas_call(
        body,
        grid=(steps,),
        in_specs=[
            pl.BlockSpec((1, blk, _D), lambda i: (0, i, 0)),
            pl.BlockSpec((1, blk, _D), lambda i: (1, i, 0)),
            pl.BlockSpec((1, 1, blk), lambda i: (i, 0, 0)),
            pl.BlockSpec((_D, _D), lambda i: (0, 0)),
            pl.BlockSpec((_D, _D), lambda i: (0, 0)),
            pl.BlockSpec((cpad, _D), lambda i: (0, 0)),
            pl.BlockSpec((1, 1, cpad), lambda i: (0, 0, 0)),
        ],
        out_specs=[
            pl.BlockSpec((_G, n_cls), lambda i: (0, 0)),
            pl.BlockSpec((_G, _D), lambda i: (0, 0)),
        ],
        out_shape=(
            jax.ShapeDtypeStruct((_G, n_cls), jnp.float32),
            jax.ShapeDtypeStruct((_G, _D), jnp.float32),
        ),
    )(ps, ps, batch3, w1, w2, w_pad, b3)


def kernel(x, edge_index, batch, conv_w, fc1_w, fc1_b):
    n, d = x.shape
    e = edge_index.shape[1]
    n_layers = conv_w.shape[0] // 2
    n_cls = fc1_w.shape[0]
    assert d == _D

    n_pad = -(-n // 128) * 128
    blk = n_pad // 4
    t_b = -(-e // (_NW * _EB))
    t_b = -(-t_b // _IC) * _IC
    e_pad = _NW * _EB * t_b

    src = edge_index[0]
    dst = edge_index[1]
    pad_n = e_pad - e
    fill = jnp.arange(pad_n, dtype=jnp.int32)
    src_p = jnp.concatenate([src, fill % n]).reshape(e_pad // _EB, _EB)
    dst_p = jnp.concatenate([dst, n + fill % (n_pad - n)]).reshape(
        e_pad // _EB, _EB)

    hz = jnp.pad(x, ((0, 2 * n_pad - n), (0, 0)))
    batch3 = jnp.concatenate(
        [batch, jnp.full((n_pad - n,), _G, jnp.int32)]).reshape(
        n_pad // blk, 1, blk)

    cpad = 16
    w_pad = jnp.pad(fc1_w, ((0, cpad - n_cls), (0, 0)))
    b3 = jnp.pad(fc1_b, (0, cpad - n_cls),
                 constant_values=-1e30).reshape(1, 1, cpad)

    seg_sum = _make_seg_sum(n_pad, t_b)
    for i in range(n_layers - 1):
        ps = seg_sum(hz, src_p, dst_p)
        hz = _mlp(ps, hz, conv_w[2 * i], conv_w[2 * i + 1], n_pad, blk)

    ps = seg_sum(hz, src_p, dst_p)
    logp, xr = _mlp_pool_head(ps, batch3, conv_w[2 * n_layers - 2],
                              conv_w[2 * n_layers - 1], w_pad, b3,
                              n_pad, blk, n_cls)
    return logp, xr


# speedup vs baseline: 1.2643x; 1.0885x over previous
import functools

import jax
import jax.numpy as jnp
from jax import lax
from jax.experimental import pallas as pl
from jax.experimental.pallas import tpu as pltpu
from jax.experimental.pallas import tpu_sc as plsc

_D = 128
_EB = 128
_IC = 40
_NC = 2
_TILES = 16
_NW = _NC * _TILES
_G = 128


def _make_seg_sum(n_pad, t_b):
    assert t_b % _IC == 0 and _IC % 8 == 0
    rows_t = n_pad // _TILES
    mesh = plsc.VectorSubcoreMesh(core_axis_name="c", subcore_axis_name="s",
                                  num_cores=_NC, num_subcores=_TILES)

    @functools.partial(
        pl.kernel,
        out_type=jax.ShapeDtypeStruct((_NC, n_pad, _D), jnp.float32),
        mesh=mesh,
        scratch_types=[
            pltpu.VMEM_SHARED((n_pad, _D), jnp.float32),
            pltpu.VMEM((_IC, _EB), jnp.int32),
            pltpu.VMEM((_IC, _EB), jnp.int32),
            pltpu.VMEM((2, _EB, _D), jnp.float32),
            pltpu.SemaphoreType.DMA((2,)),
        ],
    )
    def seg_sum(hz_hbm, src_hbm, dst_hbm, out_hbm,
                acc, src_v, dst_v, rb, sem):
        c = lax.axis_index("c")
        s = lax.axis_index("s")
        wid = s * _NC + c
        base = s * rows_t
        ebase = wid * t_b

        pltpu.sync_copy(hz_hbm.at[pl.ds(c * n_pad + base, rows_t)],
                        acc.at[pl.ds(base, rows_t)])
        plsc.subcore_barrier()

        def chunk(ci, carry):
            off = ebase + ci * _IC
            pltpu.sync_copy(src_hbm.at[pl.ds(off, _IC)], src_v)
            pltpu.sync_copy(dst_hbm.at[pl.ds(off, _IC)], dst_v)
            pltpu.async_copy(hz_hbm.at[src_v.at[0]], rb.at[0], sem.at[0])

            def body(j, carry2):
                jn = jnp.minimum(j + 1, _IC - 1)
                pltpu.async_copy(hz_hbm.at[src_v.at[jn]], rb.at[(j + 1) % 2],
                                 sem.at[(j + 1) % 2])
                pltpu.make_async_copy(hz_hbm.at[src_v.at[j]], rb.at[j % 2],
                                      sem.at[j % 2]).wait()
                pltpu.sync_copy(rb.at[j % 2], acc.at[dst_v.at[j]], add=True)
                return carry2

            lax.fori_loop(0, _IC, body, 0)
            pltpu.make_async_copy(hz_hbm.at[src_v.at[_IC - 1]], rb.at[_IC % 2],
                                  sem.at[_IC % 2]).wait()
            return carry

        lax.fori_loop(0, t_b // _IC, chunk, 0)

        plsc.subcore_barrier()
        pltpu.sync_copy(acc.at[pl.ds(base, rows_t)],
                        out_hbm.at[c].at[pl.ds(base, rows_t)])

    return seg_sum


def _mlp(ps, hz, w1, w2, n_pad, blk):
    def body(pa_ref, pb_ref, hz_ref, w1_ref, w2_ref, o_ref):
        del hz_ref
        z = pa_ref[0] + pb_ref[0]
        z = lax.dot_general(z, w1_ref[...], (((1,), (1,)), ((), ())),
                            preferred_element_type=jnp.float32,
                            precision=lax.Precision.DEFAULT)
        z = 1.0 / (1.0 + jnp.exp(-z))
        z = lax.dot_general(z, w2_ref[...], (((1,), (1,)), ((), ())),
                            preferred_element_type=jnp.float32,
                            precision=lax.Precision.DEFAULT)
        o_ref[...] = 1.0 / (1.0 + jnp.exp(-z))

    return pl.pallas_call(
        body,
        grid=(n_pad // blk,),
        in_specs=[
            pl.BlockSpec((1, blk, _D), lambda i: (0, i, 0)),
            pl.BlockSpec((1, blk, _D), lambda i: (1, i, 0)),
            pl.BlockSpec(memory_space=pl.ANY),
            pl.BlockSpec((_D, _D), lambda i: (0, 0)),
            pl.BlockSpec((_D, _D), lambda i: (0, 0)),
        ],
        out_specs=pl.BlockSpec((blk, _D), lambda i: (i, 0)),
        out_shape=jax.ShapeDtypeStruct((2 * n_pad, _D), jnp.float32),
        input_output_aliases={2: 0},
    )(ps, ps, hz, w1, w2)


def _mlp_pool_head(ps, batch3, w1, w2, w_pad, b3, n_pad, blk, n_cls):
    steps = n_pad // blk
    cpad = w_pad.shape[0]

    def body(pa_ref, pb_ref, b_ref, w1_ref, w2_ref, w_ref, bias_ref,
             logp_ref, xr_ref):
        i = pl.program_id(0)
        z = pa_ref[0] + pb_ref[0]
        z = lax.dot_general(z, w1_ref[...], (((1,), (1,)), ((), ())),
                            preferred_element_type=jnp.float32,
                            precision=lax.Precision.DEFAULT)
        z = 1.0 / (1.0 + jnp.exp(-z))
        z = lax.dot_general(z, w2_ref[...], (((1,), (1,)), ((), ())),
                            preferred_element_type=jnp.float32,
                            precision=lax.Precision.DEFAULT)
        h = 1.0 / (1.0 + jnp.exp(-z))

        bb = b_ref[0, 0, :]
        oh = (bb[:, None] == lax.broadcasted_iota(jnp.int32, (blk, _G), 1)
              ).astype(jnp.float32)
        contrib = lax.dot_general(oh, h, (((0,), (0,)), ((), ())),
                                  preferred_element_type=jnp.float32,
                                  precision=lax.Precision.HIGHEST)

        @pl.when(i == 0)
        def _():
            xr_ref[...] = contrib

        @pl.when(i > 0)
        def _():
            xr_ref[...] = xr_ref[...] + contrib

        @pl.when(i == steps - 1)
        def _():
            xr = xr_ref[...]
            logits = lax.dot_general(xr, w_ref[...], (((1,), (1,)), ((), ())),
                                     preferred_element_type=jnp.float32,
                                     precision=lax.Precision.HIGHEST)
            logits = logits + bias_ref[0, 0, :][None, :]
            m = jnp.max(logits, axis=1, keepdims=True)
            ex = jnp.exp(logits - m)
            lse = jnp.log(jnp.sum(ex, axis=1, keepdims=True))
            lp = logits - m - lse
            logp_ref[...] = lp[:, :n_cls]

    return pl.pallas_call(
        body,
        grid=(steps,),
        in_specs=[
            pl.BlockSpec((1, blk, _D), lambda i: (0, i, 0)),
            pl.BlockSpec((1, blk, _D), lambda i: (1, i, 0)),
            pl.BlockSpec((1, 1, blk), lambda i: (i, 0, 0)),
            pl.BlockSpec((_D, _D), lambda i: (0, 0)),
            pl.BlockSpec((_D, _D), lambda i: (0, 0)),
            pl.BlockSpec((cpad, _D), lambda i: (0, 0)),
            pl.BlockSpec((1, 1, cpad), lambda i: (0, 0, 0)),
        ],
        out_specs=[
            pl.BlockSpec((_G, n_cls), lambda i: (0, 0)),
            pl.BlockSpec((_G, _D), lambda i: (0, 0)),
        ],
        out_shape=(
            jax.ShapeDtypeStruct((_G, n_cls), jnp.float32),
            jax.ShapeDtypeStruct((_G, _D), jnp.float32),
        ),
    )(ps, ps, batch3, w1, w2, w_pad, b3)


def kernel(x, edge_index, batch, conv_w, fc1_w, fc1_b):
    n, d = x.shape
    e = edge_index.shape[1]
    n_layers = conv_w.shape[0] // 2
    n_cls = fc1_w.shape[0]
    assert d == _D

    n_pad = -(-n // 128) * 128
    blk = n_pad // 4
    t_b = -(-e // (_NW * _EB))
    t_b = -(-t_b // _IC) * _IC
    e_pad = _NW * _EB * t_b

    src = edge_index[0]
    dst = edge_index[1]
    pad_n = e_pad - e
    fill = jnp.arange(pad_n, dtype=jnp.int32)
    src_p = jnp.concatenate([src, fill % n]).reshape(e_pad // _EB, _EB)
    dst_p = jnp.concatenate([dst, n + fill % (n_pad - n)]).reshape(
        e_pad // _EB, _EB)

    hz = jnp.pad(x, ((0, 2 * n_pad - n), (0, 0)))
    batch3 = jnp.concatenate(
        [batch, jnp.full((n_pad - n,), _G, jnp.int32)]).reshape(
        n_pad // blk, 1, blk)

    cpad = 16
    w_pad = jnp.pad(fc1_w, ((0, cpad - n_cls), (0, 0)))
    b3 = jnp.pad(fc1_b, (0, cpad - n_cls),
                 constant_values=-1e30).reshape(1, 1, cpad)

    seg_sum = _make_seg_sum(n_pad, t_b)
    for i in range(n_layers - 1):
        ps = seg_sum(hz, src_p, dst_p)
        hz = _mlp(ps, hz, conv_w[2 * i], conv_w[2 * i + 1], n_pad, blk)

    ps = seg_sum(hz, src_p, dst_p)
    logp, xr = _mlp_pool_head(ps, batch3, conv_w[2 * n_layers - 2],
                              conv_w[2 * n_layers - 1], w_pad, b3,
                              n_pad, blk, n_cls)
    return logp, xr


# speedup vs baseline: 1.2995x; 1.0278x over previous
import functools

import jax
import jax.numpy as jnp
from jax import lax
from jax.experimental import pallas as pl
from jax.experimental.pallas import tpu as pltpu
from jax.experimental.pallas import tpu_sc as plsc

_D = 128
_EB = 128
_IC = 40
_NC = 2
_TILES = 16
_NW = _NC * _TILES
_G = 128


def _make_seg_sum(n_pad, t_b):
    assert t_b % _IC == 0 and _IC % 8 == 0
    rows_t = n_pad // _TILES
    mesh = plsc.VectorSubcoreMesh(core_axis_name="c", subcore_axis_name="s",
                                  num_cores=_NC, num_subcores=_TILES)

    @functools.partial(
        pl.kernel,
        out_type=jax.ShapeDtypeStruct((_NC, n_pad, _D), jnp.float32),
        mesh=mesh,
        scratch_types=[
            pltpu.VMEM_SHARED((n_pad, _D), jnp.float32),
            pltpu.VMEM((_IC, _EB), jnp.int32),
            pltpu.VMEM((_IC, _EB), jnp.int32),
            pltpu.VMEM((2, _EB, _D), jnp.float32),
            pltpu.SemaphoreType.DMA((2,)),
            pltpu.SemaphoreType.DMA,
        ],
    )
    def seg_sum(hz_hbm, src_hbm, dst_hbm, out_hbm,
                acc, src_v, dst_v, rb, sem, sem_i):
        c = lax.axis_index("c")
        s = lax.axis_index("s")
        wid = s * _NC + c
        base = s * rows_t
        ebase = wid * t_b

        pltpu.async_copy(hz_hbm.at[pl.ds(c * n_pad + base, rows_t)],
                         acc.at[pl.ds(base, rows_t)], sem_i)

        for ci in range(t_b // _IC):
            off = ebase + ci * _IC
            pltpu.sync_copy(src_hbm.at[pl.ds(off, _IC)], src_v)
            pltpu.sync_copy(dst_hbm.at[pl.ds(off, _IC)], dst_v)
            pltpu.async_copy(hz_hbm.at[src_v.at[0]], rb.at[0], sem.at[0])
            if ci == 0:
                pltpu.make_async_copy(
                    hz_hbm.at[pl.ds(c * n_pad + base, rows_t)],
                    acc.at[pl.ds(base, rows_t)], sem_i).wait()
                plsc.subcore_barrier()

            def body(j, carry2):
                jn = jnp.minimum(j + 1, _IC - 1)
                pltpu.async_copy(hz_hbm.at[src_v.at[jn]], rb.at[(j + 1) % 2],
                                 sem.at[(j + 1) % 2])
                pltpu.make_async_copy(hz_hbm.at[src_v.at[j]], rb.at[j % 2],
                                      sem.at[j % 2]).wait()
                pltpu.sync_copy(rb.at[j % 2], acc.at[dst_v.at[j]], add=True)
                return carry2

            lax.fori_loop(0, _IC, body, 0)
            pltpu.make_async_copy(hz_hbm.at[src_v.at[_IC - 1]], rb.at[_IC % 2],
                                  sem.at[_IC % 2]).wait()

        plsc.subcore_barrier()
        pltpu.sync_copy(acc.at[pl.ds(base, rows_t)],
                        out_hbm.at[c].at[pl.ds(base, rows_t)])

    return seg_sum


def _mlp(ps, hz, w1, w2, n_pad, blk):
    def body(pa_ref, pb_ref, hz_ref, w1_ref, w2_ref, o_ref):
        del hz_ref
        z = pa_ref[0] + pb_ref[0]
        z = lax.dot_general(z, w1_ref[...], (((1,), (1,)), ((), ())),
                            preferred_element_type=jnp.float32,
                            precision=lax.Precision.DEFAULT)
        z = 1.0 / (1.0 + jnp.exp(-z))
        z = lax.dot_general(z, w2_ref[...], (((1,), (1,)), ((), ())),
                            preferred_element_type=jnp.float32,
                            precision=lax.Precision.DEFAULT)
        o_ref[...] = 1.0 / (1.0 + jnp.exp(-z))

    return pl.pallas_call(
        body,
        grid=(n_pad // blk,),
        in_specs=[
            pl.BlockSpec((1, blk, _D), lambda i: (0, i, 0)),
            pl.BlockSpec((1, blk, _D), lambda i: (1, i, 0)),
            pl.BlockSpec(memory_space=pl.ANY),
            pl.BlockSpec((_D, _D), lambda i: (0, 0)),
            pl.BlockSpec((_D, _D), lambda i: (0, 0)),
        ],
        out_specs=pl.BlockSpec((blk, _D), lambda i: (i, 0)),
        out_shape=jax.ShapeDtypeStruct((2 * n_pad, _D), jnp.float32),
        input_output_aliases={2: 0},
    )(ps, ps, hz, w1, w2)


def _mlp_pool_head(ps, batch3, w1, w2, w_pad, b3, n_pad, blk, n_cls):
    steps = n_pad // blk
    cpad = w_pad.shape[0]

    def body(pa_ref, pb_ref, b_ref, w1_ref, w2_ref, w_ref, bias_ref,
             logp_ref, xr_ref):
        i = pl.program_id(0)
        z = pa_ref[0] + pb_ref[0]
        z = lax.dot_general(z, w1_ref[...], (((1,), (1,)), ((), ())),
                            preferred_element_type=jnp.float32,
                            precision=lax.Precision.DEFAULT)
        z = 1.0 / (1.0 + jnp.exp(-z))
        z = lax.dot_general(z, w2_ref[...], (((1,), (1,)), ((), ())),
                            preferred_element_type=jnp.float32,
                            precision=lax.Precision.DEFAULT)
        h = 1.0 / (1.0 + jnp.exp(-z))

        bb = b_ref[0, 0, :]
        oh = (bb[:, None] == lax.broadcasted_iota(jnp.int32, (blk, _G), 1)
              ).astype(jnp.float32)
        contrib = lax.dot_general(oh, h, (((0,), (0,)), ((), ())),
                                  preferred_element_type=jnp.float32,
                                  precision=lax.Precision.HIGHEST)

        @pl.when(i == 0)
        def _():
            xr_ref[...] = contrib

        @pl.when(i > 0)
        def _():
            xr_ref[...] = xr_ref[...] + contrib

        @pl.when(i == steps - 1)
        def _():
            xr = xr_ref[...]
            logits = lax.dot_general(xr, w_ref[...], (((1,), (1,)), ((), ())),
                                     preferred_element_type=jnp.float32,
                                     precision=lax.Precision.HIGHEST)
            logits = logits + bias_ref[0, 0, :][None, :]
            m = jnp.max(logits, axis=1, keepdims=True)
            ex = jnp.exp(logits - m)
            lse = jnp.log(jnp.sum(ex, axis=1, keepdims=True))
            lp = logits - m - lse
            logp_ref[...] = lp[:, :n_cls]

    return pl.pallas_call(
        body,
        grid=(steps,),
        in_specs=[
            pl.BlockSpec((1, blk, _D), lambda i: (0, i, 0)),
            pl.BlockSpec((1, blk, _D), lambda i: (1, i, 0)),
            pl.BlockSpec((1, 1, blk), lambda i: (i, 0, 0)),
            pl.BlockSpec((_D, _D), lambda i: (0, 0)),
            pl.BlockSpec((_D, _D), lambda i: (0, 0)),
            pl.BlockSpec((cpad, _D), lambda i: (0, 0)),
            pl.BlockSpec((1, 1, cpad), lambda i: (0, 0, 0)),
        ],
        out_specs=[
            pl.BlockSpec((_G, n_cls), lambda i: (0, 0)),
            pl.BlockSpec((_G, _D), lambda i: (0, 0)),
        ],
        out_shape=(
            jax.ShapeDtypeStruct((_G, n_cls), jnp.float32),
            jax.ShapeDtypeStruct((_G, _D), jnp.float32),
        ),
    )(ps, ps, batch3, w1, w2, w_pad, b3)


def kernel(x, edge_index, batch, conv_w, fc1_w, fc1_b):
    n, d = x.shape
    e = edge_index.shape[1]
    n_layers = conv_w.shape[0] // 2
    n_cls = fc1_w.shape[0]
    assert d == _D

    n_pad = -(-n // 128) * 128
    blk = n_pad // 4
    t_b = -(-e // (_NW * _EB))
    t_b = -(-t_b // _IC) * _IC
    e_pad = _NW * _EB * t_b

    src = edge_index[0]
    dst = edge_index[1]
    pad_n = e_pad - e
    fill = jnp.arange(pad_n, dtype=jnp.int32)
    src_p = jnp.concatenate([src, fill % n]).reshape(e_pad // _EB, _EB)
    dst_p = jnp.concatenate([dst, n + fill % (n_pad - n)]).reshape(
        e_pad // _EB, _EB)

    hz = jnp.pad(x, ((0, 2 * n_pad - n), (0, 0)))
    batch3 = jnp.concatenate(
        [batch, jnp.full((n_pad - n,), _G, jnp.int32)]).reshape(
        n_pad // blk, 1, blk)

    cpad = 16
    w_pad = jnp.pad(fc1_w, ((0, cpad - n_cls), (0, 0)))
    b3 = jnp.pad(fc1_b, (0, cpad - n_cls),
                 constant_values=-1e30).reshape(1, 1, cpad)

    seg_sum = _make_seg_sum(n_pad, t_b)
    for i in range(n_layers - 1):
        ps = seg_sum(hz, src_p, dst_p)
        hz = _mlp(ps, hz, conv_w[2 * i], conv_w[2 * i + 1], n_pad, blk)

    ps = seg_sum(hz, src_p, dst_p)
    logp, xr = _mlp_pool_head(ps, batch3, conv_w[2 * n_layers - 2],
                              conv_w[2 * n_layers - 1], w_pad, b3,
                              n_pad, blk, n_cls)
    return logp, xr
